# Initial kernel scaffold; baseline (speedup 1.0000x reference)
import jax
import jax.numpy as jnp
from jax.experimental import pallas as pl


def kernel(x, edge_index, W_self1, W_neigh1, b1, W_self2, W_neigh2, b2, W_fc, b_fc):
    raise NotImplementedError("write your pallas kernel here")


# speedup vs baseline: 6.6919x; 6.6919x over previous
import functools

import jax
import jax.numpy as jnp
from jax import lax
from jax.experimental import pallas as pl
from jax.experimental.pallas import tpu as pltpu
from jax.experimental.pallas import tpu_sc as plsc

N = 10000
E = 320000
H = 128
C = 40
W1 = 144

NC = 2
NS = 16
NW = NC * NS

CH = 80
CPW = E // (NW * CH)
RPS = N // NS


def _make_sc_agg(width):
  mesh = plsc.VectorSubcoreMesh(core_axis_name="c", subcore_axis_name="s")
  nv = width // 16

  @functools.partial(
      pl.kernel,
      out_type=jax.ShapeDtypeStruct((NC, N, width), jnp.float32),
      mesh=mesh,
      scratch_types=[
          pltpu.VMEM((CPW, CH), jnp.int32),
          pltpu.VMEM((CPW, CH), jnp.int32),
          pltpu.VMEM((CH, width), jnp.float32),
          pltpu.VMEM_SHARED((N, width), jnp.float32),
          pltpu.SemaphoreType.DMA,
      ],
      compiler_params=pltpu.CompilerParams(use_tc_tiling_on_sc=False),
  )
  def sc_agg(src_hbm, dst_hbm, tbl_hbm, out_hbm, src_v, dst_v, rows_v, acc, sem):
    cid = lax.axis_index("c")
    sid = lax.axis_index("s")
    wid = cid * NS + sid

    def _zero(i, carry):
      r = i // nv
      k = i % nv
      rows_v[r, pl.ds(k * 16, 16)] = jnp.zeros((16,), jnp.float32)
      return carry
    lax.fori_loop(0, CH * nv, _zero, 0)
    for k in range(RPS // CH):
      pltpu.sync_copy(rows_v, acc.at[pl.ds(sid * RPS + k * CH, CH)])
    rem = RPS % CH
    pltpu.sync_copy(rows_v.at[pl.ds(0, rem)],
                    acc.at[pl.ds(sid * RPS + RPS - rem, rem)])
    plsc.subcore_barrier()

    pltpu.sync_copy(src_hbm.at[pl.ds(wid * CPW, CPW)], src_v)
    pltpu.sync_copy(dst_hbm.at[pl.ds(wid * CPW, CPW)], dst_v)

    def _step(j, carry):
      pltpu.async_copy(tbl_hbm.at[src_v.at[j]], rows_v, sem).wait()
      pltpu.sync_copy(rows_v, acc.at[dst_v.at[j]], add=True)
      return carry
    lax.fori_loop(0, CPW, _step, 0)

    plsc.subcore_barrier()
    pltpu.sync_copy(acc.at[pl.ds(sid * RPS, RPS)],
                    out_hbm.at[cid, pl.ds(sid * RPS, RPS)])

  return sc_agg


_sc_agg_w1 = _make_sc_agg(W1)
_sc_agg_h = _make_sc_agg(H)


def _tc1_body(x_ref, aggp_ref, ws_ref, wn_ref, b_ref, h_ref, inv_ref):
  p = aggp_ref[0] + aggp_ref[1]
  deg = p[:, 128:129]
  inv = 1.0 / jnp.maximum(deg, 1.0)
  hn = p[:, :128] * inv
  h = (jnp.dot(x_ref[...], ws_ref[...], preferred_element_type=jnp.float32)
       + jnp.dot(hn, wn_ref[...], preferred_element_type=jnp.float32)
       + b_ref[...])
  h_ref[...] = jnp.maximum(h, 0.0)
  inv_ref[...] = inv


_tc1 = pl.pallas_call(
    _tc1_body,
    out_shape=(jax.ShapeDtypeStruct((N, H), jnp.float32),
               jax.ShapeDtypeStruct((N, 1), jnp.float32)),
)


def _tc2_body(h_ref, aggp_ref, inv_ref, ws_ref, wn_ref, b_ref, wfc_ref,
              bfc_ref, out_ref):
  p = aggp_ref[0] + aggp_ref[1]
  hn = p * inv_ref[...]
  z = (jnp.dot(h_ref[...], ws_ref[...], preferred_element_type=jnp.float32)
       + jnp.dot(hn, wn_ref[...], preferred_element_type=jnp.float32)
       + b_ref[...])
  out_ref[...] = (jnp.dot(z, wfc_ref[...], preferred_element_type=jnp.float32)
                  + bfc_ref[...])


_tc2 = pl.pallas_call(
    _tc2_body,
    out_shape=jax.ShapeDtypeStruct((N, C), jnp.float32),
)


def kernel(x, edge_index, W_self1, W_neigh1, b1, W_self2, W_neigh2, b2,
           W_fc, b_fc):
  src = edge_index[0].reshape(NW * CPW, CH)
  dst = edge_index[1].reshape(NW * CPW, CH)
  ones_col = jnp.ones((N, 1), jnp.float32)
  pad = jnp.zeros((N, W1 - H - 1), jnp.float32)
  x_wide = jnp.concatenate([x, ones_col, pad], axis=1)

  agg1p = _sc_agg_w1(src, dst, x_wide)
  h, inv = _tc1(x, agg1p, W_self1, W_neigh1, b1)
  agg2p = _sc_agg_h(src, dst, h)
  return _tc2(h, agg2p, inv, W_self2, W_neigh2, b2, W_fc, b_fc)


# speedup vs baseline: 9.3735x; 1.4007x over previous
import functools

import jax
import jax.numpy as jnp
from jax import lax
from jax.experimental import pallas as pl
from jax.experimental.pallas import tpu as pltpu
from jax.experimental.pallas import tpu_sc as plsc

N = 10000
E = 320000
H = 128
C = 40
W1 = 144

NC = 2
NS = 16
NW = NC * NS

RPS = N // NS


def _make_sc_agg(width, ch):
  mesh = plsc.VectorSubcoreMesh(core_axis_name="c", subcore_axis_name="s")
  nv = width // 16
  cpw = E // (NW * ch)

  @functools.partial(
      pl.kernel,
      out_type=jax.ShapeDtypeStruct((NC, N, width), jnp.float32),
      mesh=mesh,
      scratch_types=[
          pltpu.VMEM((cpw, ch), jnp.int32),
          pltpu.VMEM((cpw, ch), jnp.int32),
          pltpu.VMEM((ch, width), jnp.float32),
          pltpu.VMEM((ch, width), jnp.float32),
          pltpu.VMEM_SHARED((N, width), jnp.float32),
          pltpu.SemaphoreType.DMA,
          pltpu.SemaphoreType.DMA,
          pltpu.SemaphoreType.DMA,
          pltpu.SemaphoreType.DMA,
      ],
      compiler_params=pltpu.CompilerParams(use_tc_tiling_on_sc=False),
  )
  def sc_agg(src_hbm, dst_hbm, tbl_hbm, out_hbm, src_v, dst_v, rows_a,
             rows_b, acc, sem_a, sem_b, sem_i, sem_j):
    cid = lax.axis_index("c")
    sid = lax.axis_index("s")
    wid = cid * NS + sid

    cp_s = pltpu.async_copy(src_hbm.at[pl.ds(wid * cpw, cpw)], src_v, sem_i)
    cp_d = pltpu.async_copy(dst_hbm.at[pl.ds(wid * cpw, cpw)], dst_v, sem_j)

    def _zero(r, carry):
      for k in range(nv):
        rows_a[r, pl.ds(k * 16, 16)] = jnp.zeros((16,), jnp.float32)
      return carry
    lax.fori_loop(0, ch, _zero, 0)
    for k in range(RPS // ch):
      pltpu.sync_copy(rows_a, acc.at[pl.ds(sid * RPS + k * ch, ch)])
    rem = RPS % ch
    if rem:
      pltpu.sync_copy(rows_a.at[pl.ds(0, rem)],
                      acc.at[pl.ds(sid * RPS + RPS - rem, rem)])
    plsc.subcore_barrier()
    cp_s.wait()
    cp_d.wait()

    def _gather(j, buf, sem):
      return pltpu.async_copy(tbl_hbm.at[src_v.at[j]], buf, sem)

    def _gwait(j, buf, sem):
      pltpu.make_async_copy(tbl_hbm.at[src_v.at[j]], buf, sem).wait()

    def _scatter(j, buf):
      pltpu.sync_copy(buf, acc.at[dst_v.at[j]], add=True)

    _gather(0, rows_a, sem_a)

    def _block(k, carry):
      j = 2 * k
      _gather(j + 1, rows_b, sem_b)
      _gwait(j, rows_a, sem_a)
      _scatter(j, rows_a)
      _gather(j + 2, rows_a, sem_a)
      _gwait(j + 1, rows_b, sem_b)
      _scatter(j + 1, rows_b)
      return carry
    lax.fori_loop(0, (cpw - 1) // 2, _block, 0)
    if cpw % 2:
      _gwait(cpw - 1, rows_a, sem_a)
      _scatter(cpw - 1, rows_a)
    else:
      _gather(cpw - 1, rows_b, sem_b)
      _gwait(cpw - 2, rows_a, sem_a)
      _scatter(cpw - 2, rows_a)
      _gwait(cpw - 1, rows_b, sem_b)
      _scatter(cpw - 1, rows_b)

    plsc.subcore_barrier()
    pltpu.sync_copy(acc.at[pl.ds(sid * RPS, RPS)],
                    out_hbm.at[cid, pl.ds(sid * RPS, RPS)])

  return sc_agg


CH1 = 40
CH2 = 80
_sc_agg_w1 = _make_sc_agg(W1, CH1)
_sc_agg_h = _make_sc_agg(H, CH2)


def _tc1_body(x_ref, aggp_ref, ws_ref, wn_ref, b_ref, h_ref, inv_ref):
  p = aggp_ref[0] + aggp_ref[1]
  deg = p[:, 128:129]
  inv = 1.0 / jnp.maximum(deg, 1.0)
  hn = p[:, :128] * inv
  h = (jnp.dot(x_ref[...], ws_ref[...], preferred_element_type=jnp.float32)
       + jnp.dot(hn, wn_ref[...], preferred_element_type=jnp.float32)
       + b_ref[...])
  h_ref[...] = jnp.maximum(h, 0.0)
  inv_ref[...] = inv


_tc1 = pl.pallas_call(
    _tc1_body,
    out_shape=(jax.ShapeDtypeStruct((N, H), jnp.float32),
               jax.ShapeDtypeStruct((N, 1), jnp.float32)),
)


def _tc2_body(h_ref, aggp_ref, inv_ref, ws_ref, wn_ref, b_ref, wfc_ref,
              bfc_ref, out_ref):
  p = aggp_ref[0] + aggp_ref[1]
  hn = p * inv_ref[...]
  z = (jnp.dot(h_ref[...], ws_ref[...], preferred_element_type=jnp.float32)
       + jnp.dot(hn, wn_ref[...], preferred_element_type=jnp.float32)
       + b_ref[...])
  out_ref[...] = (jnp.dot(z, wfc_ref[...], preferred_element_type=jnp.float32)
                  + bfc_ref[...])


_tc2 = pl.pallas_call(
    _tc2_body,
    out_shape=jax.ShapeDtypeStruct((N, C), jnp.float32),
)


def kernel(x, edge_index, W_self1, W_neigh1, b1, W_self2, W_neigh2, b2,
           W_fc, b_fc):
  src1 = edge_index[0].reshape(E // CH1, CH1)
  dst1 = edge_index[1].reshape(E // CH1, CH1)
  src2 = edge_index[0].reshape(E // CH2, CH2)
  dst2 = edge_index[1].reshape(E // CH2, CH2)
  ones_col = jnp.ones((N, 1), jnp.float32)
  pad = jnp.zeros((N, W1 - H - 1), jnp.float32)
  x_wide = jnp.concatenate([x, ones_col, pad], axis=1)

  agg1p = _sc_agg_w1(src1, dst1, x_wide)
  h, inv = _tc1(x, agg1p, W_self1, W_neigh1, b1)
  agg2p = _sc_agg_h(src2, dst2, h)
  return _tc2(h, agg2p, inv, W_self2, W_neigh2, b2, W_fc, b_fc)


# speedup vs baseline: 11.7414x; 1.2526x over previous
import functools

import jax
import jax.numpy as jnp
from jax import lax
from jax.experimental import pallas as pl
from jax.experimental.pallas import tpu as pltpu
from jax.experimental.pallas import tpu_sc as plsc

N = 10000
E = 320000
H = 128
C = 40
DW = 16

NC = 2
NS = 16
NW = NC * NS

CH = 80
EPW = E // NW
CPW = EPW // CH
PH_A = (CPW + 1) // 2
PH_B = CPW - PH_A
RPS = N // NS


def _make_sc_agg(with_deg):
  mesh = plsc.VectorSubcoreMesh(core_axis_name="c", subcore_axis_name="s")

  out_type = [jax.ShapeDtypeStruct((NC, N, H), jnp.float32)]
  scratch = [
      pltpu.VMEM((PH_A * CH,), jnp.int32),
      pltpu.VMEM((PH_A * CH,), jnp.int32),
      pltpu.VMEM((CH, H), jnp.float32),
      pltpu.VMEM((CH, H), jnp.float32),
      pltpu.VMEM_SHARED((N, H), jnp.float32),
      pltpu.SemaphoreType.DMA,
      pltpu.SemaphoreType.DMA,
      pltpu.SemaphoreType.DMA,
      pltpu.SemaphoreType.DMA,
  ]
  if with_deg:
    out_type.append(jax.ShapeDtypeStruct((NC, N, DW), jnp.float32))
    scratch.append(pltpu.VMEM((CH, DW), jnp.float32))
    scratch.append(pltpu.VMEM_SHARED((N, DW), jnp.float32))

  @functools.partial(
      pl.kernel,
      out_type=tuple(out_type) if with_deg else out_type[0],
      mesh=mesh,
      scratch_types=scratch,
      compiler_params=pltpu.CompilerParams(use_tc_tiling_on_sc=False),
  )
  def sc_agg(ei_hbm, tbl_hbm, *rest):
    if with_deg:
      (out_hbm, outd_hbm, src_v, dst_v, rows_a, rows_b, acc,
       sem_a, sem_b, sem_i, sem_j, ones_v, accd) = rest
    else:
      (out_hbm, src_v, dst_v, rows_a, rows_b, acc,
       sem_a, sem_b, sem_i, sem_j) = rest
    cid = lax.axis_index("c")
    sid = lax.axis_index("s")
    wid = cid * NS + sid
    ebase = wid * EPW

    cp_s = pltpu.async_copy(ei_hbm.at[0, pl.ds(ebase, PH_A * CH)], src_v,
                            sem_i)
    cp_d = pltpu.async_copy(ei_hbm.at[1, pl.ds(ebase, PH_A * CH)], dst_v,
                            sem_j)

    def _zero(r, carry):
      for k in range(H // 16):
        rows_a[r, pl.ds(k * 16, 16)] = jnp.zeros((16,), jnp.float32)
      return carry
    lax.fori_loop(0, CH, _zero, 0)
    for k in range(RPS // CH):
      pltpu.sync_copy(rows_a, acc.at[pl.ds(sid * RPS + k * CH, CH)])
    rem = RPS % CH
    pltpu.sync_copy(rows_a.at[pl.ds(0, rem)],
                    acc.at[pl.ds(sid * RPS + RPS - rem, rem)])
    if with_deg:
      def _zod(r, carry):
        ones_v[r, pl.ds(0, DW)] = jnp.zeros((DW,), jnp.float32)
        return carry
      lax.fori_loop(0, CH, _zod, 0)
      for k in range(RPS // CH):
        pltpu.sync_copy(ones_v, accd.at[pl.ds(sid * RPS + k * CH, CH)])
      pltpu.sync_copy(ones_v.at[pl.ds(0, rem)],
                      accd.at[pl.ds(sid * RPS + RPS - rem, rem)])
      def _ones(r, carry):
        ones_v[r, pl.ds(0, DW)] = jnp.ones((DW,), jnp.float32)
        return carry
      lax.fori_loop(0, CH, _ones, 0)
    plsc.subcore_barrier()
    cp_s.wait()
    cp_d.wait()

    def _gather(j, buf, sem):
      return pltpu.async_copy(tbl_hbm.at[src_v.at[pl.ds(j * CH, CH)]], buf,
                              sem)

    def _gwait(buf, sem):
      pltpu.make_async_copy(tbl_hbm.at[src_v.at[pl.ds(0, CH)]], buf,
                            sem).wait()

    def _scatter(j, buf):
      pltpu.sync_copy(buf, acc.at[dst_v.at[pl.ds(j * CH, CH)]], add=True)
      if with_deg:
        pltpu.sync_copy(ones_v, accd.at[dst_v.at[pl.ds(j * CH, CH)]],
                        add=True)

    def _run_phase(nchunks):
      _gather(0, rows_a, sem_a)

      def _block(k, carry):
        j = 2 * k
        _gather(j + 1, rows_b, sem_b)
        _gwait(rows_a, sem_a)
        _scatter(j, rows_a)
        _gather(j + 2, rows_a, sem_a)
        _gwait(rows_b, sem_b)
        _scatter(j + 1, rows_b)
        return carry
      lax.fori_loop(0, (nchunks - 1) // 2, _block, 0)
      if nchunks % 2:
        _gwait(rows_a, sem_a)
        _scatter(nchunks - 1, rows_a)
      else:
        _gather(nchunks - 1, rows_b, sem_b)
        _gwait(rows_a, sem_a)
        _scatter(nchunks - 2, rows_a)
        _gwait(rows_b, sem_b)
        _scatter(nchunks - 1, rows_b)

    _run_phase(PH_A)
    pltpu.sync_copy(ei_hbm.at[0, pl.ds(ebase + PH_A * CH, PH_B * CH)],
                    src_v.at[pl.ds(0, PH_B * CH)])
    pltpu.sync_copy(ei_hbm.at[1, pl.ds(ebase + PH_A * CH, PH_B * CH)],
                    dst_v.at[pl.ds(0, PH_B * CH)])
    _run_phase(PH_B)

    plsc.subcore_barrier()
    pltpu.sync_copy(acc.at[pl.ds(sid * RPS, RPS)],
                    out_hbm.at[cid, pl.ds(sid * RPS, RPS)])
    if with_deg:
      pltpu.sync_copy(accd.at[pl.ds(sid * RPS, RPS)],
                      outd_hbm.at[cid, pl.ds(sid * RPS, RPS)])

  return sc_agg


_sc_agg_deg = _make_sc_agg(True)
_sc_agg = _make_sc_agg(False)


def _tc1_body(x_ref, aggp_ref, degp_ref, ws_ref, wn_ref, b_ref, h_ref,
              inv_ref):
  deg = degp_ref[0, :, 0:1] + degp_ref[1, :, 0:1]
  inv = 1.0 / jnp.maximum(deg, 1.0)
  hn = (aggp_ref[0] + aggp_ref[1]) * inv
  h = (jnp.dot(x_ref[...], ws_ref[...], preferred_element_type=jnp.float32)
       + jnp.dot(hn, wn_ref[...], preferred_element_type=jnp.float32)
       + b_ref[...])
  h_ref[...] = jnp.maximum(h, 0.0)
  inv_ref[...] = inv


_tc1 = pl.pallas_call(
    _tc1_body,
    out_shape=(jax.ShapeDtypeStruct((N, H), jnp.float32),
               jax.ShapeDtypeStruct((N, 1), jnp.float32)),
)


def _tc2_body(h_ref, aggp_ref, inv_ref, ws_ref, wn_ref, b_ref, wfc_ref,
              bfc_ref, out_ref):
  hn = (aggp_ref[0] + aggp_ref[1]) * inv_ref[...]
  z = (jnp.dot(h_ref[...], ws_ref[...], preferred_element_type=jnp.float32)
       + jnp.dot(hn, wn_ref[...], preferred_element_type=jnp.float32)
       + b_ref[...])
  out_ref[...] = (jnp.dot(z, wfc_ref[...], preferred_element_type=jnp.float32)
                  + bfc_ref[...])


_tc2 = pl.pallas_call(
    _tc2_body,
    out_shape=jax.ShapeDtypeStruct((N, C), jnp.float32),
)


def kernel(x, edge_index, W_self1, W_neigh1, b1, W_self2, W_neigh2, b2,
           W_fc, b_fc):
  agg1p, degp = _sc_agg_deg(edge_index, x)
  h, inv = _tc1(x, agg1p, degp, W_self1, W_neigh1, b1)
  agg2p = _sc_agg(edge_index, h)
  return _tc2(h, agg2p, inv, W_self2, W_neigh2, b2, W_fc, b_fc)
